# baseline (speedup 1.0000x reference)
import functools

import jax
import jax.numpy as jnp
from jax import lax
from jax.experimental import pallas as pl
from jax.experimental.pallas import tpu as pltpu
from jax.experimental.pallas import tpu_sc as plsc

_MAX_GROUP_SIZE = 4096
_CAPACITY_FACTOR = 1.25
_MIN_EXPERT_CAPACITY = 4
_NW = 32


def _groups(num_tokens, max_group_size, num_experts):
    min_num_groups = max(num_tokens // max_group_size, num_experts)
    num_groups = min_num_groups
    while num_groups < num_tokens and not (
        num_tokens % num_groups == 0 and num_groups % num_experts == 0
    ):
        num_groups += 1
    return num_groups


def _pick_stride(gc, e, row_bytes):
    max_chunk = max(8, (230 * 1024) // row_bytes)
    stride = gc + 8
    while True:
        rows = e * stride
        if rows % _NW == 0:
            per_w = rows // _NW
            if per_w % 8 == 0:
                chunk = 0
                for c in range(8, per_w + 1, 8):
                    if per_w % c == 0 and c <= max_chunk:
                        chunk = c
                if chunk:
                    return stride, chunk
        stride += 8


def _routing_body(cap, stride, x_ref, rw_ref, c0_ref, c1_ref, inv_ref,
                  gps_ref, xbf_ref):
    x = x_ref[0]
    t = x.shape[0]
    e = rw_ref.shape[1]
    ec = e * cap
    gc = cap * pl.num_programs(0)
    pad = stride - gc
    gidx = pl.program_id(0)
    xbf_ref[0] = x.astype(jnp.bfloat16)
    logits = jnp.dot(x, rw_ref[...], preferred_element_type=jnp.float32)
    probs = jax.nn.softmax(logits, axis=-1)

    eio = jax.lax.broadcasted_iota(jnp.int32, (t, e), 1)
    m0 = jnp.max(probs, axis=-1, keepdims=True)
    a0 = jnp.min(jnp.where(probs >= m0, eio, e), axis=-1, keepdims=True)
    probs1 = jnp.where(eio == a0, -1.0, probs)
    m1 = jnp.max(probs1, axis=-1, keepdims=True)
    a1 = jnp.min(jnp.where(probs1 >= m1, eio, e), axis=-1, keepdims=True)

    mask0 = (eio == a0).astype(jnp.float32)
    mask1 = (eio == a1).astype(jnp.float32)

    tr = jax.lax.broadcasted_iota(jnp.int32, (t, t), 0)
    tc = jax.lax.broadcasted_iota(jnp.int32, (t, t), 1)
    ltri = (tc <= tr).astype(jnp.float32)
    inc0 = jnp.dot(ltri, mask0, preferred_element_type=jnp.float32)
    inc1 = jnp.dot(ltri, mask1, preferred_element_type=jnp.float32)
    counts0 = jnp.sum(mask0, axis=0, keepdims=True)

    pos0 = jnp.sum(inc0 * mask0, axis=-1, keepdims=True) - 1.0
    pos1 = jnp.sum((inc1 + counts0) * mask1, axis=-1, keepdims=True) - 1.0
    pc0 = pos0.astype(jnp.int32)
    pc1 = pos1.astype(jnp.int32)
    w0 = pc0 < cap
    w1 = pc1 < cap

    tl = jax.lax.broadcasted_iota(jnp.int32, (t, 1), 0)
    tid = tl % (e * pad)
    trash = (tid // pad) * stride + gc + (tid % pad)
    base = gidx * cap
    c0_ref[0] = jnp.where(w0, a0 * stride + base + pc0, trash)
    c1_ref[0] = jnp.where(w1, a1 * stride + base + pc1, trash)

    ecio = jax.lax.broadcasted_iota(jnp.int32, (t, ec), 1)
    ej = ecio // cap
    cj = ecio - ej * cap
    d0 = ((ej == a0) & (cj == pc0) & w0).astype(jnp.float32)
    d1 = ((ej == a1) & (cj == pc1) & w1).astype(jnp.float32)
    d01 = d0 + d1
    gps_ref[0] = jnp.sum(m0 * d0 + m1 * d1, axis=0, keepdims=True)
    tcol = jax.lax.broadcasted_iota(jnp.int32, (t, ec), 0).astype(jnp.float32)
    inv_ref[0] = jnp.sum(d01 * (tcol + 1.0), axis=0,
                         keepdims=True).astype(jnp.int32)


def _dispatch_body(per_w, chunk, x_hbm, tok_hbm, buf_hbm,
                   r0, r1, i0, i1, s0, s1):
    wid = lax.axis_index("s") * 2 + lax.axis_index("c")
    base = wid * per_w
    nch = per_w // chunk
    rbufs, ibufs, sems = (r0, r1), (i0, i1), (s0, s1)
    cps = [None, None]
    pltpu.sync_copy(tok_hbm.at[pl.ds(base, chunk)], ibufs[0])
    cps[0] = pltpu.async_copy(x_hbm.at[ibufs[0]], rbufs[0], sems[0])
    for ch in range(nch):
        cur, nxt = ch % 2, (ch + 1) % 2
        if ch + 1 < nch:
            off = base + (ch + 1) * chunk
            pltpu.sync_copy(tok_hbm.at[pl.ds(off, chunk)], ibufs[nxt])
            cps[nxt] = pltpu.async_copy(x_hbm.at[ibufs[nxt]], rbufs[nxt],
                                        sems[nxt])
        cps[cur].wait()
        pltpu.sync_copy(rbufs[cur], buf_hbm.at[pl.ds(base + ch * chunk,
                                                     chunk)])


def _ffn_body(x_ref, wi_ref, wo_ref, gps_ref, y_ref):
    x = x_ref[...]
    w1 = wi_ref[0].astype(jnp.bfloat16)
    h1 = jnp.dot(x, w1, preferred_element_type=jnp.float32)
    h1 = jax.nn.gelu(h1).astype(jnp.bfloat16)
    w2 = wo_ref[0].astype(jnp.bfloat16)
    y = jnp.dot(h1, w2, preferred_element_type=jnp.float32)
    y_ref[...] = (y * gps_ref[0]).astype(jnp.bfloat16)


def _combine_body(per_w, chunk, y_hbm, c0_hbm, c1_hbm, r0_hbm, r1_hbm,
                  a0, a1, b0, b1, i0, i1, j0, j1, sa0, sa1, sb0, sb1):
    wid = lax.axis_index("s") * 2 + lax.axis_index("c")
    base = wid * per_w
    nch = per_w // chunk
    A, B, I, J = (a0, a1), (b0, b1), (i0, i1), (j0, j1)
    SA, SB = (sa0, sa1), (sb0, sb1)

    def fire(ch, r):
        off = base + ch * chunk
        pltpu.sync_copy(c0_hbm.at[pl.ds(off, chunk)], I[r])
        pltpu.sync_copy(c1_hbm.at[pl.ds(off, chunk)], J[r])
        return (pltpu.async_copy(y_hbm.at[I[r]], A[r], SA[r]),
                pltpu.async_copy(y_hbm.at[J[r]], B[r], SB[r]))

    cps = [None, None]
    cps[0] = fire(0, 0)
    for ch in range(nch):
        cur, nxt = ch % 2, (ch + 1) % 2
        if ch + 1 < nch:
            cps[nxt] = fire(ch + 1, nxt)
        cps[cur][0].wait()
        cps[cur][1].wait()
        off = base + ch * chunk
        pltpu.sync_copy(A[cur], r0_hbm.at[pl.ds(off, chunk)])
        pltpu.sync_copy(B[cur], r1_hbm.at[pl.ds(off, chunk)])


def _sum_body(a_ref, b_ref, o_ref):
    o_ref[...] = (a_ref[...].astype(jnp.float32)
                  + b_ref[...].astype(jnp.float32))


@jax.jit
def kernel(inputs, router_w, wi, wo):
    b, s, h = inputs.shape
    e = router_w.shape[1]
    f = wi.shape[2]
    num_tokens = b * s
    g = _groups(num_tokens, _MAX_GROUP_SIZE, e)
    t = num_tokens // g
    cap = max(int(round(_CAPACITY_FACTOR * t / e)), _MIN_EXPERT_CAPACITY)
    ec = e * cap
    gc = g * cap
    stride, d_chunk = _pick_stride(gc, e, h * 2)
    pad = stride - gc
    rows = e * stride

    x = inputs.reshape(g, t, h)

    c0, c1, inv, gps, xbf = pl.pallas_call(
        functools.partial(_routing_body, cap, stride),
        grid=(g,),
        in_specs=[
            pl.BlockSpec((1, t, h), lambda i: (i, 0, 0)),
            pl.BlockSpec((h, e), lambda i: (0, 0)),
        ],
        out_specs=[
            pl.BlockSpec((1, t, 1), lambda i: (i, 0, 0)),
            pl.BlockSpec((1, t, 1), lambda i: (i, 0, 0)),
            pl.BlockSpec((1, 1, ec), lambda i: (i, 0, 0)),
            pl.BlockSpec((1, 1, ec), lambda i: (i, 0, 0)),
            pl.BlockSpec((1, t, h), lambda i: (i, 0, 0)),
        ],
        out_shape=[
            jax.ShapeDtypeStruct((g, t, 1), jnp.int32),
            jax.ShapeDtypeStruct((g, t, 1), jnp.int32),
            jax.ShapeDtypeStruct((g, 1, ec), jnp.int32),
            jax.ShapeDtypeStruct((g, 1, ec), jnp.float32),
            jax.ShapeDtypeStruct((g, t, h), jnp.bfloat16),
        ],
    )(x, router_w)

    c0f = c0.reshape(num_tokens)
    c1f = c1.reshape(num_tokens)

    inv0 = inv.reshape(g, e, cap)
    gbase = (jnp.arange(g, dtype=jnp.int32) * t)[:, None, None]
    dummy = (jnp.arange(g * ec, dtype=jnp.int32).reshape(g, e, cap)
             % num_tokens)
    tok = jnp.where(inv0 > 0, inv0 - 1 + gbase, dummy)
    tok_t = tok.transpose(1, 0, 2).reshape(e, gc)
    padv = (jnp.arange(e * pad, dtype=jnp.int32).reshape(e, pad) * 17
            ) % num_tokens
    tok_row = jnp.concatenate([tok_t, padv], axis=1).reshape(rows)

    gps_t = gps.reshape(g, e, cap).transpose(1, 0, 2).reshape(e, gc)
    gps_t = jnp.pad(gps_t, ((0, 0), (0, pad))).reshape(e, stride, 1)

    h2 = h // 2
    xw = lax.bitcast_convert_type(
        xbf.reshape(num_tokens, h2, 2), jnp.int32)
    mesh = plsc.VectorSubcoreMesh(core_axis_name="c", subcore_axis_name="s")

    bufw = pl.kernel(
        functools.partial(_dispatch_body, rows // _NW, d_chunk),
        mesh=mesh,
        out_type=jax.ShapeDtypeStruct((rows, h2), jnp.int32),
        scratch_types=[
            pltpu.VMEM((d_chunk, h2), jnp.int32),
            pltpu.VMEM((d_chunk, h2), jnp.int32),
            pltpu.VMEM((d_chunk,), jnp.int32),
            pltpu.VMEM((d_chunk,), jnp.int32),
            pltpu.SemaphoreType.DMA,
            pltpu.SemaphoreType.DMA,
        ],
    )(xw, tok_row)
    buf = lax.bitcast_convert_type(bufw, jnp.bfloat16).reshape(rows, h)

    y = pl.pallas_call(
        _ffn_body,
        grid=(e,),
        in_specs=[
            pl.BlockSpec((stride, h), lambda i: (i, 0)),
            pl.BlockSpec((1, h, f), lambda i: (i, 0, 0)),
            pl.BlockSpec((1, f, h), lambda i: (i, 0, 0)),
            pl.BlockSpec((1, stride, 1), lambda i: (i, 0, 0)),
        ],
        out_specs=pl.BlockSpec((stride, h), lambda i: (i, 0)),
        out_shape=jax.ShapeDtypeStruct((rows, h), jnp.bfloat16),
    )(buf, wi, wo, gps_t)

    yw = lax.bitcast_convert_type(
        y.reshape(rows, h2, 2), jnp.int32)
    c_chunk = 32
    r0w, r1w = pl.kernel(
        functools.partial(_combine_body, num_tokens // _NW, c_chunk),
        mesh=mesh,
        out_type=[
            jax.ShapeDtypeStruct((num_tokens, h2), jnp.int32),
            jax.ShapeDtypeStruct((num_tokens, h2), jnp.int32),
        ],
        scratch_types=[
            pltpu.VMEM((c_chunk, h2), jnp.int32),
            pltpu.VMEM((c_chunk, h2), jnp.int32),
            pltpu.VMEM((c_chunk, h2), jnp.int32),
            pltpu.VMEM((c_chunk, h2), jnp.int32),
            pltpu.VMEM((c_chunk,), jnp.int32),
            pltpu.VMEM((c_chunk,), jnp.int32),
            pltpu.VMEM((c_chunk,), jnp.int32),
            pltpu.VMEM((c_chunk,), jnp.int32),
            pltpu.SemaphoreType.DMA,
            pltpu.SemaphoreType.DMA,
            pltpu.SemaphoreType.DMA,
            pltpu.SemaphoreType.DMA,
        ],
    )(yw, c0f, c1f)
    r0 = lax.bitcast_convert_type(r0w, jnp.bfloat16).reshape(num_tokens, h)
    r1 = lax.bitcast_convert_type(r1w, jnp.bfloat16).reshape(num_tokens, h)

    out = pl.pallas_call(
        _sum_body,
        grid=(g,),
        in_specs=[
            pl.BlockSpec((t, h), lambda i: (i, 0)),
            pl.BlockSpec((t, h), lambda i: (i, 0)),
        ],
        out_specs=pl.BlockSpec((t, h), lambda i: (i, 0)),
        out_shape=jax.ShapeDtypeStruct((num_tokens, h), jnp.float32),
    )(r0, r1)

    return out.reshape(b, s, h)


# speedup vs baseline: 4.4752x; 4.4752x over previous
import functools

import jax
import jax.numpy as jnp
from jax import lax
from jax.experimental import pallas as pl
from jax.experimental.pallas import tpu as pltpu
from jax.experimental.pallas import tpu_sc as plsc

_MAX_GROUP_SIZE = 4096
_CAPACITY_FACTOR = 1.25
_MIN_EXPERT_CAPACITY = 4
_NW = 32


def _groups(num_tokens, max_group_size, num_experts):
    min_num_groups = max(num_tokens // max_group_size, num_experts)
    num_groups = min_num_groups
    while num_groups < num_tokens and not (
        num_tokens % num_groups == 0 and num_groups % num_experts == 0
    ):
        num_groups += 1
    return num_groups


def _pick_chunk(per_w, row_bytes):
    max_chunk = max(8, (230 * 1024) // row_bytes)
    chunk = 8
    for c in range(8, per_w + 1, 8):
        if per_w % c == 0 and c <= max_chunk:
            chunk = c
    return chunk


def _routing_body(cap, x_ref, rw_ref, comb_ref, inv_ref):
    x = x_ref[0]
    t = x.shape[0]
    e = rw_ref.shape[1]
    ec = e * cap
    logits = jnp.dot(x, rw_ref[...], preferred_element_type=jnp.float32)
    probs = jax.nn.softmax(logits, axis=-1)

    eio = jax.lax.broadcasted_iota(jnp.int32, (t, e), 1)
    m0 = jnp.max(probs, axis=-1, keepdims=True)
    a0 = jnp.min(jnp.where(probs >= m0, eio, e), axis=-1, keepdims=True)
    probs1 = jnp.where(eio == a0, -1.0, probs)
    m1 = jnp.max(probs1, axis=-1, keepdims=True)
    a1 = jnp.min(jnp.where(probs1 >= m1, eio, e), axis=-1, keepdims=True)

    mask0 = (eio == a0).astype(jnp.float32)
    mask1 = (eio == a1).astype(jnp.float32)

    tr = jax.lax.broadcasted_iota(jnp.int32, (t, t), 0)
    tc = jax.lax.broadcasted_iota(jnp.int32, (t, t), 1)
    ltri = (tc <= tr).astype(jnp.float32)
    inc0 = jnp.dot(ltri, mask0, preferred_element_type=jnp.float32)
    inc1 = jnp.dot(ltri, mask1, preferred_element_type=jnp.float32)
    counts0 = jnp.sum(mask0, axis=0, keepdims=True)

    pos0 = jnp.sum(inc0 * mask0, axis=-1, keepdims=True) - 1.0
    pos1 = jnp.sum((inc1 + counts0) * mask1, axis=-1, keepdims=True) - 1.0
    pc0 = pos0.astype(jnp.int32)
    pc1 = pos1.astype(jnp.int32)
    w0 = pc0 < cap
    w1 = pc1 < cap

    ecio = jax.lax.broadcasted_iota(jnp.int32, (t, ec), 1)
    ej = ecio // cap
    cj = ecio - ej * cap
    d0 = ((ej == a0) & (cj == pc0) & w0).astype(jnp.float32)
    d1 = ((ej == a1) & (cj == pc1) & w1).astype(jnp.float32)

    comb_ref[0] = m0 * d0 + m1 * d1

    tcol = jax.lax.broadcasted_iota(jnp.int32, (t, ec), 0).astype(jnp.float32)
    inv_ref[0] = jnp.sum((d0 + d1) * (tcol + 1.0), axis=0,
                         keepdims=True).astype(jnp.int32)


def _dispatch_body(per_w, chunk, x_hbm, tok_hbm, buf_hbm,
                   r0, r1, i0, i1, s0, s1):
    wid = lax.axis_index("s") * 2 + lax.axis_index("c")
    base = wid * per_w
    nch = per_w // chunk
    rbufs, ibufs, sems = (r0, r1), (i0, i1), (s0, s1)
    cps = [None, None]
    pltpu.sync_copy(tok_hbm.at[pl.ds(base, chunk)], ibufs[0])
    cps[0] = pltpu.async_copy(x_hbm.at[ibufs[0]], rbufs[0], sems[0])
    for ch in range(nch):
        cur, nxt = ch % 2, (ch + 1) % 2
        if ch + 1 < nch:
            off = base + (ch + 1) * chunk
            pltpu.sync_copy(tok_hbm.at[pl.ds(off, chunk)], ibufs[nxt])
            cps[nxt] = pltpu.async_copy(x_hbm.at[ibufs[nxt]], rbufs[nxt],
                                        sems[nxt])
        cps[cur].wait()
        pltpu.sync_copy(rbufs[cur], buf_hbm.at[pl.ds(base + ch * chunk,
                                                     chunk)])


def _ffn_body(g, cap, x_ref, wi_ref, wo_ref, y_ref):
    h = x_ref.shape[-1]
    x = x_ref[...]
    h1 = jnp.dot(x, wi_ref[0], preferred_element_type=jnp.float32)
    h1 = jax.nn.gelu(h1)
    y = jnp.dot(h1, wo_ref[0], preferred_element_type=jnp.float32)
    y_ref[...] = y.reshape(g, cap, h)


def _combine_body(comb_ref, y_ref, out_ref):
    out_ref[0] = jnp.dot(comb_ref[0], y_ref[0],
                         preferred_element_type=jnp.float32)


@jax.jit
def kernel(inputs, router_w, wi, wo):
    b, s, h = inputs.shape
    e = router_w.shape[1]
    f = wi.shape[2]
    num_tokens = b * s
    g = _groups(num_tokens, _MAX_GROUP_SIZE, e)
    t = num_tokens // g
    cap = max(int(round(_CAPACITY_FACTOR * t / e)), _MIN_EXPERT_CAPACITY)
    ec = e * cap
    gc = g * cap
    rows = e * gc

    x = inputs.reshape(g, t, h)

    comb, inv = pl.pallas_call(
        functools.partial(_routing_body, cap),
        grid=(g,),
        in_specs=[
            pl.BlockSpec((1, t, h), lambda i: (i, 0, 0)),
            pl.BlockSpec((h, e), lambda i: (0, 0)),
        ],
        out_specs=[
            pl.BlockSpec((1, t, ec), lambda i: (i, 0, 0)),
            pl.BlockSpec((1, 1, ec), lambda i: (i, 0, 0)),
        ],
        out_shape=[
            jax.ShapeDtypeStruct((g, t, ec), jnp.float32),
            jax.ShapeDtypeStruct((g, 1, ec), jnp.int32),
        ],
    )(x, router_w)

    inv0 = inv.reshape(g, e, cap)
    gbase = (jnp.arange(g, dtype=jnp.int32) * t)[:, None, None]
    dummy = (jnp.arange(g * ec, dtype=jnp.int32).reshape(g, e, cap)
             % num_tokens)
    tok = jnp.where(inv0 > 0, inv0 - 1 + gbase, dummy)
    tok_row = tok.transpose(1, 0, 2).reshape(rows)

    xf = inputs.reshape(num_tokens, h)
    per_w = rows // _NW
    d_chunk = _pick_chunk(per_w, h * 4)
    mesh = plsc.VectorSubcoreMesh(core_axis_name="c", subcore_axis_name="s")

    buf = pl.kernel(
        functools.partial(_dispatch_body, per_w, d_chunk),
        mesh=mesh,
        out_type=jax.ShapeDtypeStruct((rows, h), jnp.float32),
        scratch_types=[
            pltpu.VMEM((d_chunk, h), jnp.float32),
            pltpu.VMEM((d_chunk, h), jnp.float32),
            pltpu.VMEM((d_chunk,), jnp.int32),
            pltpu.VMEM((d_chunk,), jnp.int32),
            pltpu.SemaphoreType.DMA,
            pltpu.SemaphoreType.DMA,
        ],
    )(xf, tok_row)

    y = pl.pallas_call(
        functools.partial(_ffn_body, g, cap),
        grid=(e,),
        in_specs=[
            pl.BlockSpec((gc, h), lambda i: (i, 0)),
            pl.BlockSpec((1, h, f), lambda i: (i, 0, 0)),
            pl.BlockSpec((1, f, h), lambda i: (i, 0, 0)),
        ],
        out_specs=pl.BlockSpec((g, cap, h), lambda i: (0, i, 0)),
        out_shape=jax.ShapeDtypeStruct((g, ec, h), jnp.float32),
    )(buf, wi, wo)

    out = pl.pallas_call(
        _combine_body,
        grid=(g,),
        in_specs=[
            pl.BlockSpec((1, t, ec), lambda i: (i, 0, 0)),
            pl.BlockSpec((1, ec, h), lambda i: (i, 0, 0)),
        ],
        out_specs=pl.BlockSpec((1, t, h), lambda i: (i, 0, 0)),
        out_shape=jax.ShapeDtypeStruct((g, t, h), jnp.float32),
    )(comb, y)

    return out.reshape(b, s, h)
